# baseline (speedup 1.0000x reference)
import jax
import jax.numpy as jnp
from jax.experimental import pallas as pl
from jax.experimental.pallas import tpu as pltpu

_BL = 8192


_BB = 4


def _posemb_kernel(x_ref, p_ref, w_ref, b_ref, o_ref):
    pb = p_ref[...] + b_ref[...]
    for i in range(_BB):
        y = jnp.dot(x_ref[i], w_ref[...], preferred_element_type=jnp.float32)
        o_ref[i] = y + pb


def kernel(inputs, pos_table, W, b):
    B, L, Din = inputs.shape
    Dout = W.shape[1]
    b2 = b.reshape(1, Dout)
    grid = (B // _BB,)
    return pl.pallas_call(
        _posemb_kernel,
        grid=grid,
        in_specs=[
            pl.BlockSpec((_BB, _BL, Din), lambda i: (i, 0, 0)),
            pl.BlockSpec((_BL, Dout), lambda i: (0, 0)),
            pl.BlockSpec((Din, Dout), lambda i: (0, 0)),
            pl.BlockSpec((1, Dout), lambda i: (0, 0)),
        ],
        out_specs=pl.BlockSpec((_BB, _BL, Dout), lambda i: (i, 0, 0)),
        out_shape=jax.ShapeDtypeStruct((B, L, Dout), jnp.float32),
        compiler_params=pltpu.CompilerParams(
            dimension_semantics=("parallel",),
        ),
    )(inputs, pos_table, W, b2)


# speedup vs baseline: 1.0631x; 1.0631x over previous
import jax
import jax.numpy as jnp
from jax.experimental import pallas as pl
from jax.experimental.pallas import tpu as pltpu

_BL = 8192


_BB = 1


def _posemb_kernel(x_ref, p_ref, w_ref, b_ref, o_ref):
    pb = p_ref[...] + b_ref[...]
    for i in range(_BB):
        y = jnp.dot(x_ref[i], w_ref[...], preferred_element_type=jnp.float32)
        o_ref[i] = y + pb


def kernel(inputs, pos_table, W, b):
    B, L, Din = inputs.shape
    Dout = W.shape[1]
    b2 = b.reshape(1, Dout)
    grid = (B // _BB,)
    return pl.pallas_call(
        _posemb_kernel,
        grid=grid,
        in_specs=[
            pl.BlockSpec((_BB, _BL, Din), lambda i: (i, 0, 0)),
            pl.BlockSpec((_BL, Dout), lambda i: (0, 0)),
            pl.BlockSpec((Din, Dout), lambda i: (0, 0)),
            pl.BlockSpec((1, Dout), lambda i: (0, 0)),
        ],
        out_specs=pl.BlockSpec((_BB, _BL, Dout), lambda i: (i, 0, 0)),
        out_shape=jax.ShapeDtypeStruct((B, L, Dout), jnp.float32),
        compiler_params=pltpu.CompilerParams(
            dimension_semantics=("parallel",),
        ),
    )(inputs, pos_table, W, b2)


# speedup vs baseline: 1.2147x; 1.1425x over previous
import jax
import jax.numpy as jnp
from jax.experimental import pallas as pl
from jax.experimental.pallas import tpu as pltpu

_BL = 8192


_BB = 2


def _posemb_kernel(x_ref, p_ref, w_ref, b_ref, o_ref):
    pb = p_ref[...] + b_ref[...]
    for i in range(_BB):
        y = jnp.dot(x_ref[i], w_ref[...], preferred_element_type=jnp.float32)
        o_ref[i] = y + pb


def kernel(inputs, pos_table, W, b):
    B, L, Din = inputs.shape
    Dout = W.shape[1]
    b2 = b.reshape(1, Dout)
    grid = (B // _BB,)
    return pl.pallas_call(
        _posemb_kernel,
        grid=grid,
        in_specs=[
            pl.BlockSpec((_BB, _BL, Din), lambda i: (i, 0, 0)),
            pl.BlockSpec((_BL, Dout), lambda i: (0, 0)),
            pl.BlockSpec((Din, Dout), lambda i: (0, 0)),
            pl.BlockSpec((1, Dout), lambda i: (0, 0)),
        ],
        out_specs=pl.BlockSpec((_BB, _BL, Dout), lambda i: (i, 0, 0)),
        out_shape=jax.ShapeDtypeStruct((B, L, Dout), jnp.float32),
        compiler_params=pltpu.CompilerParams(
            dimension_semantics=("parallel",),
            vmem_limit_bytes=100 * 1024 * 1024,
        ),
    )(inputs, pos_table, W, b2)


# speedup vs baseline: 1.2203x; 1.0046x over previous
import jax
import jax.numpy as jnp
from jax.experimental import pallas as pl
from jax.experimental.pallas import tpu as pltpu

_BL = 4096


def _posemb_kernel(x_ref, p_ref, w_ref, b_ref, o_ref):
    pb = p_ref[...] + b_ref[...]
    for i in range(4):
        y = jnp.dot(x_ref[i], w_ref[...], preferred_element_type=jnp.float32)
        o_ref[i] = y + pb


def kernel(inputs, pos_table, W, b):
    B, L, Din = inputs.shape
    Dout = W.shape[1]
    b2 = b.reshape(1, Dout)
    grid = (L // _BL,)
    return pl.pallas_call(
        _posemb_kernel,
        grid=grid,
        in_specs=[
            pl.BlockSpec((B, _BL, Din), lambda j: (0, j, 0)),
            pl.BlockSpec((_BL, Dout), lambda j: (j, 0)),
            pl.BlockSpec((Din, Dout), lambda j: (0, 0)),
            pl.BlockSpec((1, Dout), lambda j: (0, 0)),
        ],
        out_specs=pl.BlockSpec((B, _BL, Dout), lambda j: (0, j, 0)),
        out_shape=jax.ShapeDtypeStruct((B, L, Dout), jnp.float32),
        compiler_params=pltpu.CompilerParams(
            dimension_semantics=("parallel",),
            vmem_limit_bytes=100 * 1024 * 1024,
        ),
    )(inputs, pos_table, W, b2)
